# Initial kernel scaffold; baseline (speedup 1.0000x reference)
import jax
import jax.numpy as jnp
from jax.experimental import pallas as pl


def kernel(x, Wk, Wg):
    raise NotImplementedError("write your pallas kernel here")


# speedup vs baseline: 9.0506x; 9.0506x over previous
import functools

import jax
import jax.numpy as jnp
from jax import lax
from jax.experimental import pallas as pl
from jax.experimental.pallas import tpu as pltpu
from jax.experimental.pallas import tpu_sc as plsc

_NC, _NS = 2, 16
_NW = _NC * _NS


def _argmax_body(x_ref, wk_ref, idx_ref):
    x = x_ref[...]
    h = wk_ref.shape[0]
    xn = x / jnp.sqrt(jnp.sum(x * x, axis=1, keepdims=True))
    s = lax.dot_general(xn, wk_ref[...], (((1,), (1,)), ((), ())),
                        preferred_element_type=jnp.float32)
    m = jnp.max(s, axis=1, keepdims=True)
    hi = lax.broadcasted_iota(jnp.int32, s.shape, 1)
    idx_ref[...] = jnp.min(jnp.where(s == m, hi, jnp.int32(h)), axis=1)


def _argmax_call(x, Wk, bm):
    b, in_dim = x.shape
    h = Wk.shape[0]
    return pl.pallas_call(
        _argmax_body,
        grid=(b // bm,),
        in_specs=[
            pl.BlockSpec((bm, in_dim), lambda i: (i, 0)),
            pl.BlockSpec((h, in_dim), lambda i: (0, 0)),
        ],
        out_specs=pl.BlockSpec((bm,), lambda i: (i,)),
        out_shape=jax.ShapeDtypeStruct((b,), jnp.int32),
    )(x, Wk)


def _make_gather(h, c, b, chunk):
    b_per_w = b // _NW
    mesh = plsc.VectorSubcoreMesh(core_axis_name="c", subcore_axis_name="s")

    @functools.partial(
        pl.kernel,
        out_type=jax.ShapeDtypeStruct((b, c), jnp.float32),
        mesh=mesh,
        scratch_types=[
            pltpu.VMEM((b_per_w,), jnp.int32),
            pltpu.VMEM((chunk, c), jnp.float32),
            pltpu.SemaphoreType.DMA,
        ],
    )
    def _gather(table_hbm, idx_hbm, out_hbm, idx_v, rows_v, sem):
        wid = lax.axis_index("s") * _NC + lax.axis_index("c")
        base = wid * b_per_w
        pltpu.sync_copy(idx_hbm.at[pl.ds(base, b_per_w)], idx_v)
        for t in range(b_per_w // chunk):
            pltpu.async_copy(
                table_hbm.at[idx_v.at[pl.ds(t * chunk, chunk)]], rows_v, sem
            ).wait()
            pltpu.sync_copy(rows_v, out_hbm.at[pl.ds(base + t * chunk, chunk)])

    return _gather


def kernel(x, Wk, Wg):
    idx = _argmax_call(x, Wk, bm=256)
    c = Wg.shape[0]
    c_pad = (c + 127) // 128 * 128
    WgT = jnp.pad(Wg.T, ((0, 0), (0, c_pad - c)))
    gather = _make_gather(WgT.shape[0], c_pad, x.shape[0], chunk=64)
    return gather(WgT, idx)[:, :c]


# speedup vs baseline: 9.2600x; 1.0231x over previous
import functools

import jax
import jax.numpy as jnp
from jax import lax
from jax.experimental import pallas as pl
from jax.experimental.pallas import tpu as pltpu
from jax.experimental.pallas import tpu_sc as plsc

_NC, _NS = 2, 16
_NW = _NC * _NS


def _argmax_body(x_ref, wk_ref, idx_ref, *, kh):
    x = x_ref[...]
    bm = x.shape[0]
    h = wk_ref.shape[0]
    xn = x / jnp.sqrt(jnp.sum(x * x, axis=1, keepdims=True))
    m = jnp.full((bm, 1), -jnp.inf, jnp.float32)
    best = jnp.full((bm, 1), jnp.int32(h), jnp.int32)
    for k in range(h // kh):
        wk = wk_ref[k * kh:(k + 1) * kh, :]
        s = lax.dot_general(xn, wk, (((1,), (1,)), ((), ())),
                            preferred_element_type=jnp.float32)
        mk = jnp.max(s, axis=1, keepdims=True)
        hi = lax.broadcasted_iota(jnp.int32, s.shape, 1) + jnp.int32(k * kh)
        ik = jnp.min(jnp.where(s == mk, hi, jnp.int32(h)), axis=1,
                     keepdims=True)
        upd = mk > m
        m = jnp.where(upd, mk, m)
        best = jnp.where(upd, ik, best)
    idx_ref[...] = best[:, 0]


def _argmax_call(x, Wk, bm, kh):
    b, in_dim = x.shape
    h = Wk.shape[0]
    return pl.pallas_call(
        functools.partial(_argmax_body, kh=kh),
        grid=(b // bm,),
        in_specs=[
            pl.BlockSpec((bm, in_dim), lambda i: (i, 0)),
            pl.BlockSpec((h, in_dim), lambda i: (0, 0)),
        ],
        out_specs=pl.BlockSpec((bm,), lambda i: (i,)),
        out_shape=jax.ShapeDtypeStruct((b,), jnp.int32),
    )(x, Wk)


def _make_gather(h, c, b, chunk):
    b_per_w = b // _NW
    mesh = plsc.VectorSubcoreMesh(core_axis_name="c", subcore_axis_name="s")

    @functools.partial(
        pl.kernel,
        out_type=jax.ShapeDtypeStruct((b, c), jnp.float32),
        mesh=mesh,
        scratch_types=[
            pltpu.VMEM((b_per_w,), jnp.int32),
            pltpu.VMEM((chunk, c), jnp.float32),
            pltpu.SemaphoreType.DMA,
        ],
    )
    def _gather(table_hbm, idx_hbm, out_hbm, idx_v, rows_v, sem):
        wid = lax.axis_index("s") * _NC + lax.axis_index("c")
        base = wid * b_per_w
        pltpu.sync_copy(idx_hbm.at[pl.ds(base, b_per_w)], idx_v)
        for t in range(b_per_w // chunk):
            pltpu.async_copy(
                table_hbm.at[idx_v.at[pl.ds(t * chunk, chunk)]], rows_v, sem
            ).wait()
            pltpu.sync_copy(rows_v, out_hbm.at[pl.ds(base + t * chunk, chunk)])

    return _gather


def kernel(x, Wk, Wg):
    idx = _argmax_call(x, Wk, bm=512, kh=2048)
    c = Wg.shape[0]
    c_pad = (c + 127) // 128 * 128
    WgT = jnp.pad(Wg.T, ((0, 0), (0, c_pad - c)))
    gather = _make_gather(WgT.shape[0], c_pad, x.shape[0], chunk=64)
    return gather(WgT, idx)[:, :c]


# speedup vs baseline: 11.6040x; 1.2531x over previous
import functools

import jax
import jax.numpy as jnp
from jax import lax
from jax.experimental import pallas as pl
from jax.experimental.pallas import tpu as pltpu
from jax.experimental.pallas import tpu_sc as plsc

_NC, _NS = 2, 16
_NW = _NC * _NS


def _argmax_body(x_ref, wk_ref, wg_ref, idx_ref, wgt_ref, *, kh):
    xn = x_ref[...]
    bm = xn.shape[0]
    h = wk_ref.shape[0]
    wgt_ref[:, :wg_ref.shape[0]] = jnp.transpose(wg_ref[...])
    m = jnp.full((bm, 1), -jnp.inf, jnp.float32)
    best = jnp.full((bm, 1), jnp.float32(h), jnp.float32)
    for k in range(h // kh):
        wk = wk_ref[k * kh:(k + 1) * kh, :]
        s = lax.dot_general(xn, wk, (((1,), (1,)), ((), ())),
                            preferred_element_type=jnp.float32)
        mk = jnp.max(s, axis=1, keepdims=True)
        hi = lax.broadcasted_iota(jnp.int32, s.shape, 1).astype(jnp.float32) \
            + jnp.float32(k * kh)
        ik = jnp.min(jnp.where(s == mk, hi, jnp.float32(h)), axis=1,
                     keepdims=True)
        upd = mk > m
        m = jnp.where(upd, mk, m)
        best = jnp.where(upd, ik, best)
    idx_ref[...] = best[:, 0].astype(jnp.int32)


def _argmax_call(x, Wk, Wg, c_pad, bm, kh):
    b, in_dim = x.shape
    h = Wk.shape[0]
    c = Wg.shape[0]
    grid = b // bm
    hs = h // grid
    return pl.pallas_call(
        functools.partial(_argmax_body, kh=kh),
        grid=(grid,),
        in_specs=[
            pl.BlockSpec((bm, in_dim), lambda i: (i, 0)),
            pl.BlockSpec((h, in_dim), lambda i: (0, 0)),
            pl.BlockSpec((c, hs), lambda i: (0, i)),
        ],
        out_specs=[
            pl.BlockSpec((bm,), lambda i: (i,)),
            pl.BlockSpec((hs, c_pad), lambda i: (i, 0)),
        ],
        out_shape=[
            jax.ShapeDtypeStruct((b,), jnp.int32),
            jax.ShapeDtypeStruct((h, c_pad), jnp.float32),
        ],
    )(x, Wk, Wg)


def _make_gather(h, c_pad, c_out, b, chunk):
    b_per_w = b // _NW
    n = b_per_w // chunk
    mesh = plsc.VectorSubcoreMesh(core_axis_name="c", subcore_axis_name="s")

    @functools.partial(
        pl.kernel,
        out_type=jax.ShapeDtypeStruct((b, c_out), jnp.float32),
        mesh=mesh,
        scratch_types=[
            pltpu.VMEM((b_per_w,), jnp.int32),
            pltpu.VMEM((chunk, c_pad), jnp.float32),
            pltpu.VMEM((chunk, c_pad), jnp.float32),
            pltpu.SemaphoreType.DMA,
            pltpu.SemaphoreType.DMA,
            pltpu.SemaphoreType.DMA,
            pltpu.SemaphoreType.DMA,
        ],
    )
    def _gather(table_hbm, idx_hbm, out_hbm, idx_v, rows0, rows1,
                g0, g1, s0, s1):
        wid = lax.axis_index("s") * _NC + lax.axis_index("c")
        base = wid * b_per_w
        pltpu.sync_copy(idx_hbm.at[pl.ds(base, b_per_w)], idx_v)
        bufs, gsem, ssem = (rows0, rows1), (g0, g1), (s0, s1)
        gh = [None] * n
        sh = [None] * n
        gh[0] = pltpu.async_copy(
            table_hbm.at[idx_v.at[pl.ds(0, chunk)]], bufs[0], gsem[0])
        for t in range(n):
            cur = t & 1
            if t + 1 < n:
                if t >= 1:
                    sh[t - 1].wait()
                gh[t + 1] = pltpu.async_copy(
                    table_hbm.at[idx_v.at[pl.ds((t + 1) * chunk, chunk)]],
                    bufs[(t + 1) & 1], gsem[(t + 1) & 1])
            gh[t].wait()
            src = bufs[cur] if c_out == c_pad \
                else bufs[cur].at[:, pl.ds(0, c_out)]
            sh[t] = pltpu.async_copy(
                src, out_hbm.at[pl.ds(base + t * chunk, chunk)], ssem[cur])
        sh[n - 2].wait()
        sh[n - 1].wait()

    return _gather


def kernel(x, Wk, Wg):
    xn = x / jnp.linalg.norm(x, axis=1, keepdims=True)
    c = Wg.shape[0]
    c_pad = (c + 127) // 128 * 128
    idx, WgT = _argmax_call(xn, Wk, Wg, c_pad, bm=512, kh=1024)
    gather = _make_gather(WgT.shape[0], c_pad, c_pad, x.shape[0], chunk=32)
    return gather(WgT, idx)[:, :c]
